# baseline (speedup 1.0000x reference)
import functools

import jax
import jax.numpy as jnp
from jax import lax
from jax.experimental import pallas as pl
from jax.experimental.pallas import tpu as pltpu
from jax.experimental.pallas import tpu_sc as plsc

NC, NS, L = 2, 16, 16
NW = NC * NS
CNT_H, CNT_X = 5, 14
MAXB = 14
MAXS = 3
SDIMS = (2, 3, 3, 3)
CHN = 64
TROWS = 4 * MAXB
BLK = 128


def _cnt_of(vec):
    return jnp.where(vec == 1, CNT_H, CNT_X).astype(jnp.int32)


def _main_body(tch, nch, nv, t_real, an_ref, cf_ref, emb_ref, out_ref,
               an_v, b_v, cf_v, emb_v, tab_v, stage_v, sem0, sem1):
    wid = lax.axis_index("s") * NC + lax.axis_index("c")
    s0 = wid * tch

    pltpu.sync_copy(emb_ref, emb_v)
    zeros = jnp.zeros((L,), jnp.float32)
    for t in range(4):
        for p in range(MAXB):
            for g in range(CHN // L):
                if p < SDIMS[t]:
                    val = emb_v[pl.ds(t * MAXS * CHN + p * CHN + g * L, L)]
                else:
                    val = zeros
                tab_v[pl.ds((t * MAXB + p) * CHN + g * L, L)] = val

    pltpu.sync_copy(an_ref, an_v)
    pltpu.sync_copy(cf_ref.at[pl.ds(s0, tch)], cf_v)

    CO = 8

    def blk_total(m):
        acc = jnp.zeros((L,), jnp.int32)
        for u in range(CO):
            acc = acc + _cnt_of(an_v[pl.ds((m * CO + u) * L, L)])
        return jnp.sum(acc)

    def c_cond(st):
        _, off, ct = st
        return off + ct <= s0

    def c_body(st):
        m, off, ct = st
        return (m + 1, off + ct, blk_total(m + 1))

    m_c, off_c, _ = lax.while_loop(c_cond, c_body, (0, 0, blk_total(0)))

    def vec_total(k):
        return jnp.sum(_cnt_of(an_v[pl.ds(k * L, L)]))

    def f_cond(st):
        _, off, ct = st
        return off + ct <= s0

    def f_body(st):
        k, off, ct = st
        return (k + 1, off + ct, vec_total(k + 1))

    k0 = m_c * CO
    k_f, off_f, _ = lax.while_loop(f_cond, f_body, (k0, off_c, vec_total(k0)))
    nb = k_f * L

    def p2_body(j, off):
        vec = an_v[pl.ds((k_f + j) * L, L)]
        cnt = _cnt_of(vec)
        aidx = jnp.where(vec == 1, 0,
                         jnp.where(vec == 6, 1,
                                   jnp.where(vec == 7, 2, 3))).astype(jnp.int32)
        cs = plsc.cumsum(cnt)
        b_v[pl.ds(j * L, L)] = aidx * MAXB - (off + cs - cnt)
        return off + jnp.sum(cnt)

    lax.fori_loop(0, nv, p2_body, off_f)

    iota16 = lax.iota(jnp.int32, L)
    loc_max = nv * L - 1
    sems = (sem0, sem1)

    def st_copy(j, slot):
        return pltpu.make_async_copy(
            stage_v.at[slot],
            out_ref.at[:, pl.ds(s0 + j * BLK, BLK)], sems[slot])

    def build(j, slot):
        keys = []
        for v in range(BLK // L):
            c = cf_v[pl.ds(j * BLK + v * L, L)]
            loc = jnp.clip(c - nb, 0, loc_max)
            bg = plsc.load_gather(b_v, [loc])
            key = bg + (s0 + j * BLK + v * L) + iota16
            keys.append(jnp.clip(key, 0, TROWS - 1) * CHN)
        for ch in range(CHN):
            for v in range(BLK // L):
                gat = plsc.load_gather(tab_v, [keys[v] + ch])
                stage_v[slot, ch, pl.ds(v * L, L)] = gat

    ncw = jnp.minimum(nch, (t_real - s0 + BLK - 1) // BLK)
    npair = ncw // 2
    odd = ncw - 2 * npair

    def pair_body(g, carry):
        @pl.when(g > 0)
        def _():
            st_copy(0, 0).wait()
        build(2 * g, 0)
        st_copy(2 * g, 0).start()

        @pl.when(g > 0)
        def _():
            st_copy(0, 1).wait()
        build(2 * g + 1, 1)
        st_copy(2 * g + 1, 1).start()
        return carry

    lax.fori_loop(0, npair, pair_body, 0)

    @pl.when(odd == 1)
    def _():
        @pl.when(npair > 0)
        def _():
            st_copy(0, 0).wait()
        build(ncw - 1, 0)
        st_copy(ncw - 1, 0).start()

    st_copy(0, 0).wait()

    @pl.when(npair > 0)
    def _():
        st_copy(0, 1).wait()


def kernel(atomic_numbers, coeff_ind_to_node_ind, emb_weight):
    n = atomic_numbers.shape[0]
    t = coeff_ind_to_node_ind.shape[0]
    tch = -(-t // (NW * BLK)) * BLK
    tpad = NW * tch
    nch = tch // BLK
    nv = (tch // CNT_H + 64) // L + 2
    anp = -(-n // L) * L + nv * L

    an_p = jnp.pad(atomic_numbers.astype(jnp.int32), (0, anp - n),
                   constant_values=8)
    cf_p = jnp.pad(coeff_ind_to_node_ind.astype(jnp.int32), (0, tpad - t))

    mesh = plsc.VectorSubcoreMesh(core_axis_name="c", subcore_axis_name="s")
    cparams = pltpu.CompilerParams(needs_layout_passes=False,
                                   use_tc_tiling_on_sc=True)

    out_k = pl.kernel(
        functools.partial(_main_body, tch, nch, nv, t),
        out_type=jax.ShapeDtypeStruct((CHN, t), jnp.float32),
        mesh=mesh,
        compiler_params=cparams,
        scratch_types=[
            pltpu.VMEM((anp,), jnp.int32),
            pltpu.VMEM(((nv + 1) * L,), jnp.int32),
            pltpu.VMEM((tch,), jnp.int32),
            pltpu.VMEM((4 * MAXS * CHN,), jnp.float32),
            pltpu.VMEM((TROWS * CHN,), jnp.float32),
            pltpu.VMEM((2, CHN, BLK), jnp.float32),
            pltpu.SemaphoreType.DMA,
            pltpu.SemaphoreType.DMA,
        ],
    )(an_p, cf_p, emb_weight.reshape(-1))
    return out_k.T


# speedup vs baseline: 1.9863x; 1.9863x over previous
import functools

import jax
import jax.numpy as jnp
from jax import lax
from jax.experimental import pallas as pl
from jax.experimental.pallas import tpu as pltpu
from jax.experimental.pallas import tpu_sc as plsc

NC, NS, L = 2, 16, 16
NW = NC * NS
CNT_H, CNT_X = 5, 14
MAXB = 14
MAXS = 3
SDIMS = (2, 3, 3, 3)
CHN = 64
TROWS = 4 * MAXB
BLK = 128
TSTR = CHN + 1


def _cnt_of(vec):
    return jnp.where(vec == 1, CNT_H, CNT_X).astype(jnp.int32)


def _main_body(tch, nch, nv, t_real, an_ref, cf_ref, emb_ref, out_ref,
               an_v, b_v, cf_v, emb_v, tab_v, stage_v, sem0, sem1):
    wid = lax.axis_index("s") * NC + lax.axis_index("c")
    s0 = wid * tch

    pltpu.sync_copy(emb_ref, emb_v)
    zeros = jnp.zeros((L,), jnp.float32)
    for t in range(4):
        for p in range(MAXB):
            for g in range(CHN // L):
                if p < SDIMS[t]:
                    val = emb_v[pl.ds(t * MAXS * CHN + p * CHN + g * L, L)]
                else:
                    val = zeros
                tab_v[pl.ds((t * MAXB + p) * TSTR + g * L, L)] = val

    pltpu.sync_copy(an_ref, an_v)
    pltpu.sync_copy(cf_ref.at[pl.ds(s0, tch)], cf_v)

    CO = 8

    def blk_total(m):
        acc = jnp.zeros((L,), jnp.int32)
        for u in range(CO):
            acc = acc + _cnt_of(an_v[pl.ds((m * CO + u) * L, L)])
        return jnp.sum(acc)

    def c_cond(st):
        _, off, ct = st
        return off + ct <= s0

    def c_body(st):
        m, off, ct = st
        return (m + 1, off + ct, blk_total(m + 1))

    m_c, off_c, _ = lax.while_loop(c_cond, c_body, (0, 0, blk_total(0)))

    def vec_total(k):
        return jnp.sum(_cnt_of(an_v[pl.ds(k * L, L)]))

    def f_cond(st):
        _, off, ct = st
        return off + ct <= s0

    def f_body(st):
        k, off, ct = st
        return (k + 1, off + ct, vec_total(k + 1))

    k0 = m_c * CO
    k_f, off_f, _ = lax.while_loop(f_cond, f_body, (k0, off_c, vec_total(k0)))
    nb = k_f * L

    def p2_body(j, off):
        vec = an_v[pl.ds((k_f + j) * L, L)]
        cnt = _cnt_of(vec)
        aidx = jnp.where(vec == 1, 0,
                         jnp.where(vec == 6, 1,
                                   jnp.where(vec == 7, 2, 3))).astype(jnp.int32)
        cs = plsc.cumsum(cnt)
        b_v[pl.ds(j * L, L)] = aidx * MAXB - (off + cs - cnt)
        return off + jnp.sum(cnt)

    lax.fori_loop(0, nv, p2_body, off_f)

    iota16 = lax.iota(jnp.int32, L)
    loc_max = nv * L - 1
    sems = (sem0, sem1)

    def st_copy(j, slot):
        return pltpu.make_async_copy(
            stage_v.at[slot],
            out_ref.at[:, pl.ds(s0 + j * BLK, BLK)], sems[slot])

    def build(j, slot):
        keys = []
        for v in range(BLK // L):
            c = cf_v[pl.ds(j * BLK + v * L, L)]
            loc = jnp.clip(c - nb, 0, loc_max)
            bg = plsc.load_gather(b_v, [loc])
            key = bg + (s0 + j * BLK + v * L) + iota16
            keys.append(jnp.clip(key, 0, TROWS - 1) * TSTR)
        for ch in range(CHN):
            for v in range(BLK // L):
                gat = plsc.load_gather(tab_v, [keys[v] + ch])
                stage_v[slot, ch, pl.ds(v * L, L)] = gat

    ncw = jnp.minimum(nch, (t_real - s0 + BLK - 1) // BLK)
    npair = ncw // 2
    odd = ncw - 2 * npair

    def pair_body(g, carry):
        @pl.when(g > 0)
        def _():
            st_copy(0, 0).wait()
        build(2 * g, 0)
        st_copy(2 * g, 0).start()

        @pl.when(g > 0)
        def _():
            st_copy(0, 1).wait()
        build(2 * g + 1, 1)
        st_copy(2 * g + 1, 1).start()
        return carry

    lax.fori_loop(0, npair, pair_body, 0)

    @pl.when(odd == 1)
    def _():
        @pl.when(npair > 0)
        def _():
            st_copy(0, 0).wait()
        build(ncw - 1, 0)
        st_copy(ncw - 1, 0).start()

    st_copy(0, 0).wait()

    @pl.when(npair > 0)
    def _():
        st_copy(0, 1).wait()


def kernel(atomic_numbers, coeff_ind_to_node_ind, emb_weight):
    n = atomic_numbers.shape[0]
    t = coeff_ind_to_node_ind.shape[0]
    tch = -(-t // (NW * BLK)) * BLK
    tpad = NW * tch
    nch = tch // BLK
    nv = (tch // CNT_H + 64) // L + 2
    anp = -(-n // L) * L + nv * L

    an_p = jnp.pad(atomic_numbers.astype(jnp.int32), (0, anp - n),
                   constant_values=8)
    cf_p = jnp.pad(coeff_ind_to_node_ind.astype(jnp.int32), (0, tpad - t))

    mesh = plsc.VectorSubcoreMesh(core_axis_name="c", subcore_axis_name="s")
    cparams = pltpu.CompilerParams(needs_layout_passes=False,
                                   use_tc_tiling_on_sc=True)

    out_k = pl.kernel(
        functools.partial(_main_body, tch, nch, nv, t),
        out_type=jax.ShapeDtypeStruct((CHN, t), jnp.float32),
        mesh=mesh,
        compiler_params=cparams,
        scratch_types=[
            pltpu.VMEM((anp,), jnp.int32),
            pltpu.VMEM(((nv + 1) * L,), jnp.int32),
            pltpu.VMEM((tch,), jnp.int32),
            pltpu.VMEM((4 * MAXS * CHN,), jnp.float32),
            pltpu.VMEM((TROWS * TSTR + L,), jnp.float32),
            pltpu.VMEM((2, CHN, BLK), jnp.float32),
            pltpu.SemaphoreType.DMA,
            pltpu.SemaphoreType.DMA,
        ],
    )(an_p, cf_p, emb_weight.reshape(-1))
    return out_k.T


# speedup vs baseline: 3.8626x; 1.9446x over previous
import functools

import jax
import jax.numpy as jnp
from jax import lax
from jax.experimental import pallas as pl
from jax.experimental.pallas import tpu as pltpu
from jax.experimental.pallas import tpu_sc as plsc

NC, NS, L = 2, 16, 16
NW = NC * NS
CNT_H, CNT_X = 5, 14
MAXB = 14
MAXS = 3
SDIMS = (2, 3, 3, 3)
CHN = 64
TROWS = 4 * MAXB
BLK = 128
TSTR = CHN + 1


def _cnt_of(vec):
    return jnp.where(vec == 1, CNT_H, CNT_X).astype(jnp.int32)


def _main_body(tch, nch, nv, t_real, an_ref, cf_ref, emb_ref, out_ref,
               an_v, b_v, cf_v, emb_v, tab_v, stage_v, sem0, sem1):
    wid = lax.axis_index("s") * NC + lax.axis_index("c")
    s0 = wid * tch

    pltpu.sync_copy(emb_ref, emb_v)
    zeros = jnp.zeros((L,), jnp.float32)
    for t in range(4):
        for p in range(MAXB):
            for g in range(CHN // L):
                if p < SDIMS[t]:
                    val = emb_v[pl.ds(t * MAXS * CHN + p * CHN + g * L, L)]
                else:
                    val = zeros
                tab_v[pl.ds((t * MAXB + p) * TSTR + g * L, L)] = val

    pltpu.sync_copy(an_ref, an_v)
    pltpu.sync_copy(cf_ref.at[pl.ds(s0, tch)], cf_v)

    CO = 8

    def blk_total(m):
        acc = jnp.zeros((L,), jnp.int32)
        for u in range(CO):
            acc = acc + _cnt_of(an_v[pl.ds((m * CO + u) * L, L)])
        return jnp.sum(acc)

    def c_cond(st):
        _, off, ct = st
        return off + ct <= s0

    def c_body(st):
        m, off, ct = st
        return (m + 1, off + ct, blk_total(m + 1))

    m_c, off_c, _ = lax.while_loop(c_cond, c_body, (0, 0, blk_total(0)))

    def vec_total(k):
        return jnp.sum(_cnt_of(an_v[pl.ds(k * L, L)]))

    def f_cond(st):
        _, off, ct = st
        return off + ct <= s0

    def f_body(st):
        k, off, ct = st
        return (k + 1, off + ct, vec_total(k + 1))

    k0 = m_c * CO
    k_f, off_f, _ = lax.while_loop(f_cond, f_body, (k0, off_c, vec_total(k0)))
    nb = k_f * L

    def p2_body(j, off):
        vec = an_v[pl.ds((k_f + j) * L, L)]
        cnt = _cnt_of(vec)
        aidx = jnp.where(vec == 1, 0,
                         jnp.where(vec == 6, 1,
                                   jnp.where(vec == 7, 2, 3))).astype(jnp.int32)
        cs = plsc.cumsum(cnt)
        b_v[pl.ds(j * L, L)] = aidx * MAXB - (off + cs - cnt)
        return off + jnp.sum(cnt)

    lax.fori_loop(0, nv, p2_body, off_f)

    iota16 = lax.iota(jnp.int32, L)
    loc_max = nv * L - 1
    sems = (sem0, sem1)

    def st_copy(j, slot):
        return pltpu.make_async_copy(
            stage_v.at[slot],
            out_ref.at[:, pl.ds(s0 + j * BLK, BLK)], sems[slot])

    def build(j, slot):
        keys = []
        for v in range(BLK // L):
            c = cf_v[pl.ds(j * BLK + v * L, L)]
            loc = jnp.clip(c - nb, 0, loc_max)
            bg = plsc.load_gather(b_v, [loc])
            key = bg + (s0 + j * BLK + v * L) + iota16
            keys.append(jnp.clip(key, 0, TROWS - 1) * TSTR)
        for ch in range(CHN):
            gs = [plsc.load_gather(tab_v, [keys[v] + ch])
                  for v in range(BLK // L)]
            for v in range(BLK // L):
                stage_v[slot, ch, pl.ds(v * L, L)] = gs[v]

    ncw = jnp.minimum(nch, (t_real - s0 + BLK - 1) // BLK)
    npair = ncw // 2
    odd = ncw - 2 * npair

    def pair_body(g, carry):
        @pl.when(g > 0)
        def _():
            st_copy(0, 0).wait()
        build(2 * g, 0)
        st_copy(2 * g, 0).start()

        @pl.when(g > 0)
        def _():
            st_copy(0, 1).wait()
        build(2 * g + 1, 1)
        st_copy(2 * g + 1, 1).start()
        return carry

    lax.fori_loop(0, npair, pair_body, 0)

    @pl.when(odd == 1)
    def _():
        @pl.when(npair > 0)
        def _():
            st_copy(0, 0).wait()
        build(ncw - 1, 0)
        st_copy(ncw - 1, 0).start()

    st_copy(0, 0).wait()

    @pl.when(npair > 0)
    def _():
        st_copy(0, 1).wait()


def kernel(atomic_numbers, coeff_ind_to_node_ind, emb_weight):
    n = atomic_numbers.shape[0]
    t = coeff_ind_to_node_ind.shape[0]
    tch = -(-t // (NW * BLK)) * BLK
    tpad = NW * tch
    nch = tch // BLK
    nv = (tch // CNT_H + 64) // L + 2
    anp = -(-n // L) * L + nv * L

    an_p = jnp.pad(atomic_numbers.astype(jnp.int32), (0, anp - n),
                   constant_values=8)
    cf_p = jnp.pad(coeff_ind_to_node_ind.astype(jnp.int32), (0, tpad - t))

    mesh = plsc.VectorSubcoreMesh(core_axis_name="c", subcore_axis_name="s")
    cparams = pltpu.CompilerParams(needs_layout_passes=False,
                                   use_tc_tiling_on_sc=True)

    out_k = pl.kernel(
        functools.partial(_main_body, tch, nch, nv, t),
        out_type=jax.ShapeDtypeStruct((CHN, t), jnp.float32),
        mesh=mesh,
        compiler_params=cparams,
        scratch_types=[
            pltpu.VMEM((anp,), jnp.int32),
            pltpu.VMEM(((nv + 1) * L,), jnp.int32),
            pltpu.VMEM((tch,), jnp.int32),
            pltpu.VMEM((4 * MAXS * CHN,), jnp.float32),
            pltpu.VMEM((TROWS * TSTR + L,), jnp.float32),
            pltpu.VMEM((2, CHN, BLK), jnp.float32),
            pltpu.SemaphoreType.DMA,
            pltpu.SemaphoreType.DMA,
        ],
    )(an_p, cf_p, emb_weight.reshape(-1))
    return out_k.T
